# baseline (speedup 1.0000x reference)
import functools

import jax
import jax.numpy as jnp
from jax.experimental import pallas as pl
from jax.experimental.pallas import tpu as pltpu


def _cdiv(a, b):
    return -(-a // b)


_NQ = 4


def _ffn_kernel(x_ref, w1_hbm, b1_ref, w2_hbm, b2_ref, o_ref, w1_v, w2_v,
                h_buf, sem):
    hidden = w1_v.shape[1]
    q = hidden // _NQ
    i = pl.program_id(0)
    n_tiles = pl.num_programs(0) - 1
    slot = jax.lax.rem(i, 2)
    prev_slot = jax.lax.rem(i + 1, 2)

    @pl.when(i == 0)
    def _():
        for k in range(_NQ):
            sl = slice(k * q, (k + 1) * q)
            pltpu.make_async_copy(w1_hbm.at[:, sl], w1_v.at[:, sl], sem.at[k]).start()
        for k in range(_NQ):
            sl = slice(k * q, (k + 1) * q)
            pltpu.make_async_copy(w2_hbm.at[sl], w2_v.at[sl], sem.at[_NQ + k]).start()

        x = x_ref[...]
        for k in range(_NQ):
            sl = slice(k * q, (k + 1) * q)
            pltpu.make_async_copy(w1_v.at[:, sl], w1_v.at[:, sl], sem.at[k]).wait()
            hk = jnp.dot(x, w1_v[:, sl], preferred_element_type=jnp.float32)
            h_buf[0, :, sl] = jnp.maximum(hk + b1_ref[:, sl], 0.0)

    @pl.when(i == 1)
    def _():
        h = jnp.dot(x_ref[...], w1_v[...], preferred_element_type=jnp.float32)
        h_buf[1] = jnp.maximum(h + b1_ref[...], 0.0)
        y = b2_ref[...]
        for k in range(_NQ):
            sl = slice(k * q, (k + 1) * q)
            pltpu.make_async_copy(w2_v.at[sl], w2_v.at[sl], sem.at[_NQ + k]).wait()
            y = y + jnp.dot(h_buf[0, :, sl], w2_v[sl, :],
                            preferred_element_type=jnp.float32)
        o_ref[...] = y

    @pl.when(jnp.logical_and(i >= 2, i < n_tiles))
    def _():
        h = jnp.dot(x_ref[...], w1_v[...], preferred_element_type=jnp.float32)
        h_buf[slot] = jnp.maximum(h + b1_ref[...], 0.0)
        y = jnp.dot(h_buf[prev_slot], w2_v[...], preferred_element_type=jnp.float32)
        o_ref[...] = y + b2_ref[...]

    @pl.when(i == n_tiles)
    def _():
        y = jnp.dot(h_buf[prev_slot], w2_v[...], preferred_element_type=jnp.float32)
        o_ref[...] = y + b2_ref[...]


@functools.partial(jax.jit, static_argnames=("block_m",))
def _ffn(x, w1, b1, w2, b2, *, block_m=512):
    batch, seq, d_model = x.shape
    hidden = w1.shape[1]
    M = batch * seq

    x2d = x.reshape(M, d_model)
    bm = min(block_m, M)
    n_m = _cdiv(M, bm)
    last = n_m - 1

    out2d = pl.pallas_call(
        _ffn_kernel,
        out_shape=jax.ShapeDtypeStruct((M, d_model), jnp.float32),
        grid=(n_m + 1,),
        in_specs=[
            pl.BlockSpec((bm, d_model),
                         lambda i: (jnp.minimum(i, last), 0)),
            pl.BlockSpec(memory_space=pl.ANY),
            pl.BlockSpec((1, hidden), lambda i: (0, 0)),
            pl.BlockSpec(memory_space=pl.ANY),
            pl.BlockSpec((1, d_model), lambda i: (0, 0)),
        ],
        out_specs=pl.BlockSpec((bm, d_model),
                               lambda i: (jnp.maximum(i - 1, 0), 0)),
        scratch_shapes=[
            pltpu.VMEM((d_model, hidden), jnp.float32),
            pltpu.VMEM((hidden, d_model), jnp.float32),
            pltpu.VMEM((2, bm, hidden), jnp.float32),
            pltpu.SemaphoreType.DMA((2 * _NQ,)),
        ],
        compiler_params=pltpu.CompilerParams(
            dimension_semantics=("arbitrary",),
            vmem_limit_bytes=int(0.98 * 64 * 1024 * 1024),
        ),
    )(x2d, w1, b1, w2, b2)

    return out2d.reshape(batch, seq, d_model)


def kernel(x, w1, b1, w2, b2):
    return _ffn(x, w1, b1, w2, b2)


# speedup vs baseline: 1.0342x; 1.0342x over previous
import functools

import jax
import jax.numpy as jnp
from jax.experimental import pallas as pl
from jax.experimental.pallas import tpu as pltpu


def _cdiv(a, b):
    return -(-a // b)


_NQ1 = 8
_NQ2 = 4


def _ffn_kernel(x_ref, w1_hbm, b1_ref, w2_hbm, b2_ref, o_ref, w1_v, w2_v, sem):
    hidden = w1_v.shape[1]
    q1 = hidden // _NQ1
    q2 = hidden // _NQ2
    first = pl.program_id(0) == 0

    @pl.when(first)
    def _():
        for k in range(_NQ1):
            sl = slice(k * q1, (k + 1) * q1)
            pltpu.make_async_copy(w1_hbm.at[:, sl], w1_v.at[:, sl], sem.at[k]).start()
        for k in range(_NQ2):
            sl = slice(k * q2, (k + 1) * q2)
            pltpu.make_async_copy(w2_hbm.at[sl], w2_v.at[sl],
                                  sem.at[_NQ1 + k]).start()

        x = x_ref[...]
        hs = []
        for k in range(_NQ1):
            sl = slice(k * q1, (k + 1) * q1)
            pltpu.make_async_copy(w1_v.at[:, sl], w1_v.at[:, sl], sem.at[k]).wait()
            hk = jnp.dot(x, w1_v[:, sl], preferred_element_type=jnp.float32)
            hs.append(jnp.maximum(hk + b1_ref[:, sl], 0.0))
        h = jnp.concatenate(hs, axis=1)
        y = b2_ref[...]
        for k in range(_NQ2):
            sl = slice(k * q2, (k + 1) * q2)
            pltpu.make_async_copy(w2_v.at[sl], w2_v.at[sl], sem.at[_NQ1 + k]).wait()
            y = y + jnp.dot(h[:, sl], w2_v[sl, :], preferred_element_type=jnp.float32)
        o_ref[...] = y

    @pl.when(jnp.logical_not(first))
    def _():
        h = jnp.dot(x_ref[...], w1_v[...], preferred_element_type=jnp.float32)
        h = jnp.maximum(h + b1_ref[...], 0.0)
        y = jnp.dot(h, w2_v[...], preferred_element_type=jnp.float32)
        o_ref[...] = y + b2_ref[...]


@functools.partial(jax.jit, static_argnames=("block_m",))
def _ffn(x, w1, b1, w2, b2, *, block_m=512):
    batch, seq, d_model = x.shape
    hidden = w1.shape[1]
    M = batch * seq

    x2d = x.reshape(M, d_model)
    bm = min(block_m, M)
    n_m = _cdiv(M, bm)

    out2d = pl.pallas_call(
        _ffn_kernel,
        out_shape=jax.ShapeDtypeStruct((M, d_model), jnp.float32),
        grid=(n_m,),
        in_specs=[
            pl.BlockSpec((bm, d_model), lambda i: (i, 0)),
            pl.BlockSpec(memory_space=pl.ANY),
            pl.BlockSpec((1, hidden), lambda i: (0, 0)),
            pl.BlockSpec(memory_space=pl.ANY),
            pl.BlockSpec((1, d_model), lambda i: (0, 0)),
        ],
        out_specs=pl.BlockSpec((bm, d_model), lambda i: (i, 0)),
        scratch_shapes=[
            pltpu.VMEM((d_model, hidden), jnp.float32),
            pltpu.VMEM((hidden, d_model), jnp.float32),
            pltpu.SemaphoreType.DMA((_NQ1 + _NQ2,)),
        ],
        compiler_params=pltpu.CompilerParams(
            dimension_semantics=("arbitrary",),
            vmem_limit_bytes=int(0.95 * 64 * 1024 * 1024),
        ),
    )(x2d, w1, b1, w2, b2)

    return out2d.reshape(batch, seq, d_model)


def kernel(x, w1, b1, w2, b2):
    return _ffn(x, w1, b1, w2, b2)
